# Initial kernel scaffold; baseline (speedup 1.0000x reference)
import jax
import jax.numpy as jnp
from jax.experimental import pallas as pl


def kernel(x, edge_index, edge_values, W0, W1, offset0, scale0, offset1, scale1):
    raise NotImplementedError("write your pallas kernel here")


# speedup vs baseline: 4.5418x; 4.5418x over previous
import functools

import jax
import jax.numpy as jnp
from jax import lax
from jax.experimental import pallas as pl
from jax.experimental.pallas import tpu as pltpu
from jax.experimental.pallas import tpu_sc as plsc

N = 10000
E = 320000
D = 128

NUM_CORES = 2
NUM_SUBCORES = 16
NUM_WORKERS = NUM_CORES * NUM_SUBCORES
EDGES_PER_WORKER = E // NUM_WORKERS
CHUNK = 80
NUM_CHUNKS = EDGES_PER_WORKER // CHUNK
NPAD = 10240
ROWS_PER_TILE = NPAD // NUM_SUBCORES
ZCHUNK = 128
NZ = ROWS_PER_TILE // ZCHUNK


def _spmm_body(x_hbm, src_hbm, dst_hbm, w_hbm, out_hbm,
               acc, src_v, dst_v, w_v, rows_v, zbuf, sem):
    cid = lax.axis_index("core")
    sid = lax.axis_index("subcore")
    wid = cid * NUM_SUBCORES + sid

    @pl.loop(0, ZCHUNK)
    def _(i):
        for k in range(D // 16):
            zbuf[i, pl.ds(k * 16, 16)] = jnp.zeros((16,), jnp.float32)

    row0 = sid * ROWS_PER_TILE
    for z in range(NZ):
        pltpu.sync_copy(zbuf, acc.at[pl.ds(row0 + z * ZCHUNK, ZCHUNK)])
    plsc.subcore_barrier()

    base0 = wid * EDGES_PER_WORKER

    @pl.loop(0, NUM_CHUNKS)
    def _(j):
        base = base0 + j * CHUNK
        pltpu.sync_copy(src_hbm.at[pl.ds(base, CHUNK)], src_v)
        pltpu.sync_copy(dst_hbm.at[pl.ds(base, CHUNK)], dst_v)
        pltpu.sync_copy(w_hbm.at[pl.ds(base, CHUNK)], w_v)
        pltpu.async_copy(x_hbm.at[src_v], rows_v, sem).wait()

        @pl.loop(0, CHUNK // 16)
        def _(g):
            wg = w_v[pl.ds(g * 16, 16)]
            for e in range(16):
                we = wg[e]
                r = g * 16 + e
                for k in range(D // 16):
                    sl = pl.ds(k * 16, 16)
                    rows_v[r, sl] = rows_v[r, sl] * we

        pltpu.sync_copy(rows_v, acc.at[dst_v], add=True)

    plsc.subcore_barrier()

    for z in range(NZ):
        r = row0 + z * ZCHUNK
        pltpu.sync_copy(acc.at[pl.ds(r, ZCHUNK)], out_hbm.at[cid].at[pl.ds(r, ZCHUNK)])


def _sc_spmm(x, src, dst, w):
    mesh = plsc.VectorSubcoreMesh(core_axis_name="core", subcore_axis_name="subcore")
    kern = pl.kernel(
        _spmm_body,
        out_type=jax.ShapeDtypeStruct((NUM_CORES, NPAD, D), jnp.float32),
        mesh=mesh,
        scratch_types=[
            pltpu.VMEM_SHARED((NPAD, D), jnp.float32),
            pltpu.VMEM((CHUNK,), jnp.int32),
            pltpu.VMEM((CHUNK,), jnp.int32),
            pltpu.VMEM((CHUNK,), jnp.float32),
            pltpu.VMEM((CHUNK, D), jnp.float32),
            pltpu.VMEM((ZCHUNK, D), jnp.float32),
            pltpu.SemaphoreType.DMA,
        ],
    )
    return kern(x, src, dst, w)


def _bn_relu(vw, scale, offset):
    mean = jnp.mean(vw, axis=1, keepdims=True)
    cent = vw - mean
    var = jnp.mean(cent * cent, axis=1, keepdims=True)
    return jnp.maximum(scale * cent * lax.rsqrt(var + 1e-9) + offset, 0.0)


def _tc_body(x_ref, p0_ref, p1_ref, w0_ref, w1_ref, off0_ref, sc0_ref,
             off1_ref, sc1_ref, out_ref):
    h0 = x_ref[...]
    h1 = p0_ref[0] + p1_ref[0]
    vw0 = jnp.dot(h0, w0_ref[...], preferred_element_type=jnp.float32)
    vw1 = jnp.dot(h1, w1_ref[...], preferred_element_type=jnp.float32)
    v0 = _bn_relu(vw0, sc0_ref[...], off0_ref[...])
    v1 = _bn_relu(vw1, sc1_ref[...], off1_ref[...])
    out_ref[...] = v0 + v1


def _tc_fused(x, partials, W0, W1, offset0, scale0, offset1, scale1):
    blk = 1000
    grid = (N // blk,)
    row_spec = pl.BlockSpec((blk, D), lambda i: (i, 0))
    full = pl.BlockSpec((D, D), lambda i: (0, 0))
    vec = pl.BlockSpec((1, D), lambda i: (0, 0))
    return pl.pallas_call(
        _tc_body,
        grid=grid,
        in_specs=[row_spec,
                  pl.BlockSpec((1, blk, D), lambda i: (0, i, 0)),
                  pl.BlockSpec((1, blk, D), lambda i: (1, i, 0)),
                  full, full, vec, vec, vec, vec],
        out_specs=row_spec,
        out_shape=jax.ShapeDtypeStruct((N, D), jnp.float32),
    )(x, partials, partials, W0, W1, offset0, scale0, offset1, scale1)


def kernel(x, edge_index, edge_values, W0, W1, offset0, scale0, offset1, scale1):
    dst = edge_index[0].astype(jnp.int32)
    src = edge_index[1].astype(jnp.int32)
    partials = _sc_spmm(x, src, dst, edge_values)
    return _tc_fused(x, partials, W0, W1, offset0, scale0, offset1, scale1)


# speedup vs baseline: 11.2151x; 2.4693x over previous
import jax
import jax.numpy as jnp
from jax import lax
from jax.experimental import pallas as pl
from jax.experimental.pallas import tpu as pltpu
from jax.experimental.pallas import tpu_sc as plsc

N = 10000
E = 320000
D = 128

NUM_CORES = 2
NUM_SUBCORES = 16
NUM_WORKERS = NUM_CORES * NUM_SUBCORES
EDGES_PER_WORKER = E // NUM_WORKERS
CHUNK = 80
NUM_CHUNKS = EDGES_PER_WORKER // CHUNK
NPAD = 10240
ROWS_PER_TILE = NPAD // NUM_SUBCORES


def _spmm_body(x_hbm, src_hbm, dst_hbm, w_hbm, out_hbm,
               acc, src_v, dst0, dst1, w0, w1, rows0, rows1, sem0, sem1):
    cid = lax.axis_index("core")
    sid = lax.axis_index("subcore")
    wid = cid * NUM_SUBCORES + sid
    rbufs = (rows0, rows1)
    dbufs = (dst0, dst1)
    wbufs = (w0, w1)
    sems = (sem0, sem1)
    base0 = wid * EDGES_PER_WORKER

    pltpu.make_async_copy(src_hbm.at[pl.ds(base0, EDGES_PER_WORKER)],
                          src_v, sem0).start()

    @pl.loop(0, CHUNK)
    def _(i):
        for k in range(D // 16):
            rows1[i, pl.ds(k * 16, 16)] = jnp.zeros((16,), jnp.float32)

    row0 = sid * ROWS_PER_TILE
    for z in range(ROWS_PER_TILE // CHUNK):
        pltpu.sync_copy(rows1, acc.at[pl.ds(row0 + z * CHUNK, CHUNK)])
    plsc.subcore_barrier()

    pltpu.make_async_copy(src_hbm.at[pl.ds(base0, EDGES_PER_WORKER)],
                          src_v, sem0).wait()

    def chunk_copies(j, b):
        base = base0 + j * CHUNK
        return (
            pltpu.make_async_copy(x_hbm.at[src_v.at[pl.ds(j * CHUNK, CHUNK)]],
                                  rbufs[b], sems[b]),
            pltpu.make_async_copy(dst_hbm.at[pl.ds(base, CHUNK)], dbufs[b], sems[b]),
            pltpu.make_async_copy(w_hbm.at[pl.ds(base, CHUNK)], wbufs[b], sems[b]),
        )

    def start(j, b):
        for cp in chunk_copies(j, b):
            cp.start()

    def step(j, b, last):
        if not last:
            start(j + 1, 1 - b)
        for cp in chunk_copies(j, b):
            cp.wait()
        rows, wv = rbufs[b], wbufs[b]

        @pl.loop(0, CHUNK // 16)
        def _(g):
            wg = wv[pl.ds(g * 16, 16)]
            for e in range(16):
                we = wg[e]
                r = g * 16 + e
                for k in range(D // 16):
                    sl = pl.ds(k * 16, 16)
                    rows[r, sl] = rows[r, sl] * we

        pltpu.sync_copy(rows, acc.at[dbufs[b]], add=True)

    start(0, 0)

    @pl.loop(0, NUM_CHUNKS - 1, step=2)
    def _(j):
        step(j, 0, False)
        step(j + 1, 1, False)

    step(NUM_CHUNKS - 1, 0, True)

    plsc.subcore_barrier()

    pltpu.sync_copy(acc.at[pl.ds(row0, ROWS_PER_TILE)],
                    out_hbm.at[cid].at[pl.ds(row0, ROWS_PER_TILE)])


def _sc_spmm(x, src, dst, w):
    mesh = plsc.VectorSubcoreMesh(core_axis_name="core", subcore_axis_name="subcore")
    kern = pl.kernel(
        _spmm_body,
        out_type=jax.ShapeDtypeStruct((NUM_CORES, NPAD, D), jnp.float32),
        mesh=mesh,
        scratch_types=[
            pltpu.VMEM_SHARED((NPAD, D), jnp.float32),
            pltpu.VMEM((EDGES_PER_WORKER,), jnp.int32),
            pltpu.VMEM((CHUNK,), jnp.int32),
            pltpu.VMEM((CHUNK,), jnp.int32),
            pltpu.VMEM((CHUNK,), jnp.float32),
            pltpu.VMEM((CHUNK,), jnp.float32),
            pltpu.VMEM((CHUNK, D), jnp.float32),
            pltpu.VMEM((CHUNK, D), jnp.float32),
            pltpu.SemaphoreType.DMA,
            pltpu.SemaphoreType.DMA,
        ],
    )
    return kern(x, src, dst, w)


def _bn_relu(vw, scale, offset):
    mean = jnp.mean(vw, axis=1, keepdims=True)
    cent = vw - mean
    var = jnp.mean(cent * cent, axis=1, keepdims=True)
    return jnp.maximum(scale * cent * lax.rsqrt(var + 1e-9) + offset, 0.0)


def _tc_body(x_ref, p0_ref, p1_ref, w0_ref, w1_ref, off0_ref, sc0_ref,
             off1_ref, sc1_ref, out_ref):
    h0 = x_ref[...]
    h1 = p0_ref[0] + p1_ref[0]
    vw0 = jnp.dot(h0, w0_ref[...], preferred_element_type=jnp.float32)
    vw1 = jnp.dot(h1, w1_ref[...], preferred_element_type=jnp.float32)
    v0 = _bn_relu(vw0, sc0_ref[...], off0_ref[...])
    v1 = _bn_relu(vw1, sc1_ref[...], off1_ref[...])
    out_ref[...] = v0 + v1


def _tc_fused(x, partials, W0, W1, offset0, scale0, offset1, scale1):
    blk = 1000
    grid = (N // blk,)
    row_spec = pl.BlockSpec((blk, D), lambda i: (i, 0))
    full = pl.BlockSpec((D, D), lambda i: (0, 0))
    vec = pl.BlockSpec((1, D), lambda i: (0, 0))
    return pl.pallas_call(
        _tc_body,
        grid=grid,
        in_specs=[row_spec,
                  pl.BlockSpec((1, blk, D), lambda i: (0, i, 0)),
                  pl.BlockSpec((1, blk, D), lambda i: (1, i, 0)),
                  full, full, vec, vec, vec, vec],
        out_specs=row_spec,
        out_shape=jax.ShapeDtypeStruct((N, D), jnp.float32),
    )(x, partials, partials, W0, W1, offset0, scale0, offset1, scale1)


def kernel(x, edge_index, edge_values, W0, W1, offset0, scale0, offset1, scale1):
    dst = edge_index[0].astype(jnp.int32)
    src = edge_index[1].astype(jnp.int32)
    partials = _sc_spmm(x, src, dst, edge_values)
    return _tc_fused(x, partials, W0, W1, offset0, scale0, offset1, scale1)
